# baseline (speedup 1.0000x reference)
import functools

import jax
import jax.numpy as jnp
from jax import lax
from jax.experimental import pallas as pl
from jax.experimental.pallas import tpu as pltpu
from jax.experimental.pallas import tpu_sc as plsc

_NC = 2
_NS = 16
_L = 16
_NW = _NC * _NS

_ROWS = 128
_COLS = 32768
_RPW = _ROWS // _NW
_CW = 4096
_NCHUNK = _COLS // _CW
_UNROLL = 4
_NEG = float("-inf")


def _insert5(stack, v):
    out = []
    for t in range(5):
        hi = jnp.maximum(stack[t], v)
        v = jnp.minimum(stack[t], v)
        out.append(hi)
    return out


_GV = 32
_GROUPS = _CW // (_GV * _L)
_NGRP = _NCHUNK * _RPW * _GROUPS


_CGRP = _RPW * _GROUPS
_CHE = _RPW * _CW


def _sc_body(x_hbm, out_hbm, buf0, buf1, gsum, cand, obuf, sem0, sem1):
    wid = lax.axis_index("s") * _NC + lax.axis_index("c")
    row0 = wid * _RPW

    bufs = (buf0, buf1)
    sems = (sem0, sem1)

    neg = jnp.full((_L,), _NEG, dtype=jnp.float32)
    iota = lax.iota(jnp.int32, _L)

    def dyn_start(kk, h):
        for j in range(_RPW):
            pltpu.make_async_copy(
                x_hbm.at[row0 + j, pl.ds(kk * _CW, _CW)],
                bufs[h].at[pl.ds(j * _CW, _CW)], sems[h]).start()

    def dyn_wait(kk, h):
        for j in range(_RPW):
            pltpu.make_async_copy(
                x_hbm.at[row0 + j, pl.ds(kk * _CW, _CW)],
                bufs[h].at[pl.ds(j * _CW, _CW)], sems[h]).wait()

    dyn_start(0, 0)
    dyn_start(1, 1)

    def pair(it, carry):
        for h in range(2):
            kk = it * 2 + h
            buf = bufs[h]
            dyn_wait(kk, h)

            @plsc.parallel_loop(0, _CGRP, unroll=1, carry=neg)
            def sm_chunk(i, c, buf=buf):
                base = i * _GV * _L
                vs = [buf[pl.ds(base + t * _L, _L)] for t in range(_GV)]
                while len(vs) > 1:
                    vs = [jnp.maximum(vs[p], vs[p + 1])
                          for p in range(0, len(vs) - 1, 2)] + (
                              [vs[-1]] if len(vs) % 2 else [])
                gsum[pl.ds(i * _L, _L)] = vs[0]
                return jnp.maximum(c, vs[0])

            m_run = jnp.maximum(carry[0], sm_chunk)
            srt = jnp.sort(m_run)
            thr = jnp.max(jnp.where(iota == _L - 5, srt, _NEG))
            hit = jnp.any(sm_chunk >= thr)

            def docollect(_):
                def cstep(q, p):
                    m = gsum[pl.ds(q * _L, _L)]
                    h2 = jnp.any(m >= thr)
                    cand[p] = q
                    return p + h2.astype(jnp.int32)
                return lax.fori_loop(0, _CGRP, cstep, jnp.int32(0))

            p_k = lax.cond(hit, docollect, lambda _: jnp.int32(0), 0)

            def rstep(c, f, buf=buf):
                base = cand[c] * (_GV * _L)

                def ustep(u, ff, buf=buf, base=base):
                    fl = list(ff)
                    for w in range(_UNROLL):
                        v = buf[pl.ds(base + (u * _UNROLL + w) * _L, _L)]
                        fl[w * 5:(w + 1) * 5] = _insert5(
                            fl[w * 5:(w + 1) * 5], v)
                    return tuple(fl)

                return lax.fori_loop(0, _GV // _UNROLL, ustep, f)

            F = lax.fori_loop(0, p_k, rstep, carry[1:])

            @pl.when(kk + 2 < _NCHUNK)
            def _(kk=kk, h=h):
                dyn_start(kk + 2, h)

            carry = (m_run,) + tuple(F)
        return carry

    carry = lax.fori_loop(0, _NCHUNK // 2, pair,
                          (neg,) + tuple(neg for _ in range(5 * _UNROLL)))
    F = carry[1:]

    merged = list(F[0:5])
    for w in range(1, _UNROLL):
        for t in range(5):
            merged = _insert5(merged, F[w * 5 + t])

    for t in range(5):
        obuf[pl.ds(t * _L, _L)] = merged[t]
    pltpu.sync_copy(obuf, out_hbm.at[wid])


@jax.jit
def _sc_topk_candidates(x):
    mesh = plsc.VectorSubcoreMesh(core_axis_name="c", subcore_axis_name="s",
                                  num_cores=_NC, num_subcores=_NS)
    k = pl.kernel(
        _sc_body,
        out_type=jax.ShapeDtypeStruct((_NW, 5 * _L), jnp.float32),
        mesh=mesh,
        scratch_types=[
            pltpu.VMEM((_CHE,), jnp.float32),
            pltpu.VMEM((_CHE,), jnp.float32),
            pltpu.VMEM((_CGRP * _L,), jnp.float32),
            pltpu.SMEM((_CGRP,), jnp.int32),
            pltpu.VMEM((5 * _L,), jnp.float32),
            pltpu.SemaphoreType.DMA,
            pltpu.SemaphoreType.DMA,
        ],
        compiler_params=pltpu.CompilerParams(needs_layout_passes=False),
    )
    return k(x)


def _merge_body(c_ref, yt_ref, o_ref):
    x = c_ref[...]
    r, l = x.shape
    li = (lax.broadcasted_iota(jnp.int32, (r, l), 0) * l
          + lax.broadcasted_iota(jnp.int32, (r, l), 1))
    prod = jnp.float32(1.0)
    for _ in range(5):
        t = jnp.max(x)
        sel = jnp.where(x == t, li, jnp.int32(2 ** 30))
        fi = jnp.min(sel)
        x = jnp.where(li == fi, _NEG, x)
        prod = prod * (jnp.float32(1.0) - t)
    y_site = jnp.float32(1.0) - prod
    d = y_site - yt_ref[0, 0]
    o_ref[0, 0] = d * d


@jax.jit
def _merge_loss(cands, y_true):
    return pl.pallas_call(
        _merge_body,
        out_shape=jax.ShapeDtypeStruct((1, 1), jnp.float32),
        in_specs=[
            pl.BlockSpec(memory_space=pltpu.VMEM),
            pl.BlockSpec(memory_space=pltpu.SMEM),
        ],
        out_specs=pl.BlockSpec(memory_space=pltpu.SMEM),
    )(cands, y_true)


def kernel(y_pred, y_true):
    cands = _sc_topk_candidates(y_pred)
    loss = _merge_loss(cands, y_true.reshape(1, 1))
    return loss.reshape(1)


# speedup vs baseline: 1.0653x; 1.0653x over previous
import jax
import jax.numpy as jnp
from jax import lax
from jax.experimental import pallas as pl
from jax.experimental.pallas import tpu as pltpu
from jax.experimental.pallas import tpu_sc as plsc

_NC = 2
_NS = 16
_L = 16
_NW = _NC * _NS

_ROWS = 128
_COLS = 32768
_SCROWS = 32
_TCROWS = _ROWS - _SCROWS
_RPW = _SCROWS // _NW
_CW = 16384
_NCHUNK = _COLS // _CW
_UNROLL = 4
_NEG = float("-inf")

_GV = 32
_CGRP = _RPW * _CW // (_GV * _L)
_CHE = _RPW * _CW


def _insert5(stack, v):
    out = []
    for t in range(5):
        hi = jnp.maximum(stack[t], v)
        v = jnp.minimum(stack[t], v)
        out.append(hi)
    return out


def _sc_body(x_hbm, out_hbm, buf0, buf1, gsum, cand, obuf, sem0, sem1):
    wid = lax.axis_index("s") * _NC + lax.axis_index("c")
    row0 = _TCROWS + wid * _RPW

    bufs = (buf0, buf1)
    sems = (sem0, sem1)

    neg = jnp.full((_L,), _NEG, dtype=jnp.float32)
    iota = lax.iota(jnp.int32, _L)

    def dyn_start(kk, h):
        for j in range(_RPW):
            pltpu.make_async_copy(
                x_hbm.at[row0 + j, pl.ds(kk * _CW, _CW)],
                bufs[h].at[pl.ds(j * _CW, _CW)], sems[h]).start()

    def dyn_wait(kk, h):
        for j in range(_RPW):
            pltpu.make_async_copy(
                x_hbm.at[row0 + j, pl.ds(kk * _CW, _CW)],
                bufs[h].at[pl.ds(j * _CW, _CW)], sems[h]).wait()

    dyn_start(0, 0)
    dyn_start(1, 1)

    def pair(it, carry):
        for h in range(2):
            kk = it * 2 + h
            buf = bufs[h]
            dyn_wait(kk, h)

            @plsc.parallel_loop(0, _CGRP, unroll=1, carry=neg)
            def sm_chunk(i, c, buf=buf):
                base = i * _GV * _L
                vs = [buf[pl.ds(base + t * _L, _L)] for t in range(_GV)]
                while len(vs) > 1:
                    vs = [jnp.maximum(vs[p], vs[p + 1])
                          for p in range(0, len(vs) - 1, 2)] + (
                              [vs[-1]] if len(vs) % 2 else [])
                gsum[pl.ds(i * _L, _L)] = vs[0]
                return jnp.maximum(c, vs[0])

            m_run = jnp.maximum(carry[0], sm_chunk)
            srt = jnp.sort(m_run)
            thr = jnp.max(jnp.where(iota == _L - 5, srt, _NEG))
            hit = jnp.any(sm_chunk >= thr)

            def docollect(_):
                def cstep(q, p):
                    m = gsum[pl.ds(q * _L, _L)]
                    h2 = jnp.any(m >= thr)
                    cand[p] = q
                    return p + h2.astype(jnp.int32)
                return lax.fori_loop(0, _CGRP, cstep, jnp.int32(0))

            p_k = lax.cond(hit, docollect, lambda _: jnp.int32(0), 0)

            def rstep(c, f, buf=buf):
                base = cand[c] * (_GV * _L)

                def ustep(u, ff, buf=buf, base=base):
                    fl = list(ff)
                    for w in range(_UNROLL):
                        v = buf[pl.ds(base + (u * _UNROLL + w) * _L, _L)]
                        fl[w * 5:(w + 1) * 5] = _insert5(
                            fl[w * 5:(w + 1) * 5], v)
                    return tuple(fl)

                return lax.fori_loop(0, _GV // _UNROLL, ustep, f)

            F = lax.fori_loop(0, p_k, rstep, carry[1:])

            @pl.when(kk + 2 < _NCHUNK)
            def _(kk=kk, h=h):
                dyn_start(kk + 2, h)

            carry = (m_run,) + tuple(F)
        return carry

    carry = lax.fori_loop(0, _NCHUNK // 2, pair,
                          (neg,) + tuple(neg for _ in range(5 * _UNROLL)))
    F = carry[1:]

    merged = list(F[0:5])
    for w in range(1, _UNROLL):
        for t in range(5):
            merged = _insert5(merged, F[w * 5 + t])

    for t in range(5):
        obuf[pl.ds(t * _L, _L)] = merged[t]
    pltpu.sync_copy(obuf, out_hbm.at[wid])


def _sc_topk_candidates(x):
    mesh = plsc.VectorSubcoreMesh(core_axis_name="c", subcore_axis_name="s",
                                  num_cores=_NC, num_subcores=_NS)
    k = pl.kernel(
        _sc_body,
        out_type=jax.ShapeDtypeStruct((_NW, 5 * _L), jnp.float32),
        mesh=mesh,
        scratch_types=[
            pltpu.VMEM((_CHE,), jnp.float32),
            pltpu.VMEM((_CHE,), jnp.float32),
            pltpu.VMEM((_CGRP * _L,), jnp.float32),
            pltpu.SMEM((_CGRP,), jnp.int32),
            pltpu.VMEM((5 * _L,), jnp.float32),
            pltpu.SemaphoreType.DMA,
            pltpu.SemaphoreType.DMA,
        ],
        compiler_params=pltpu.CompilerParams(needs_layout_passes=False),
    )
    return k(x)


_TCB = 4096
_TCG = _COLS // _TCB


def _tc_body(x_ref, o_ref, acc):
    @pl.when(pl.program_id(0) == 0)
    def _init():
        acc[...] = jnp.full((5, 8, 128), _NEG, jnp.float32)

    A = tuple(acc[t] for t in range(5))

    def cstep(c, a):
        a = list(a)
        for r in range(_TCROWS // 8):
            v = x_ref[pl.ds(r * 8, 8), pl.ds(c * 128, 128)]
            a = _insert5(a, v)
        return tuple(a)

    A = lax.fori_loop(0, _TCB // 128, cstep, A)
    for t in range(5):
        acc[t] = A[t]

    @pl.when(pl.program_id(0) == pl.num_programs(0) - 1)
    def _out():
        o_ref[...] = acc[...]


def _tc_topk_candidates(x):
    return pl.pallas_call(
        _tc_body,
        grid=(_TCG,),
        in_specs=[pl.BlockSpec((_TCROWS, _TCB), lambda i: (0, i))],
        out_specs=pl.BlockSpec((5, 8, 128), lambda i: (0, 0, 0)),
        out_shape=jax.ShapeDtypeStruct((5, 8, 128), jnp.float32),
        scratch_shapes=[pltpu.VMEM((5, 8, 128), jnp.float32)],
    )(x)


def _merge_body(c1_ref, c2_ref, yt_ref, o_ref):
    x1 = c1_ref[...]
    x2 = c2_ref[...]
    r1, l1 = x1.shape
    r2, l2 = x2.shape
    li1 = (lax.broadcasted_iota(jnp.int32, (r1, l1), 0) * l1
           + lax.broadcasted_iota(jnp.int32, (r1, l1), 1))
    li2 = (lax.broadcasted_iota(jnp.int32, (r2, l2), 0) * l2
           + lax.broadcasted_iota(jnp.int32, (r2, l2), 1) + r1 * l1)
    big = jnp.int32(2 ** 30)
    prod = jnp.float32(1.0)
    for _ in range(5):
        t = jnp.maximum(jnp.max(x1), jnp.max(x2))
        fi = jnp.minimum(jnp.min(jnp.where(x1 == t, li1, big)),
                         jnp.min(jnp.where(x2 == t, li2, big)))
        x1 = jnp.where(li1 == fi, _NEG, x1)
        x2 = jnp.where(li2 == fi, _NEG, x2)
        prod = prod * (jnp.float32(1.0) - t)
    y_site = jnp.float32(1.0) - prod
    d = y_site - yt_ref[0, 0]
    o_ref[0, 0] = d * d


def _merge_loss(c1, c2, y_true):
    return pl.pallas_call(
        _merge_body,
        out_shape=jax.ShapeDtypeStruct((1, 1), jnp.float32),
        in_specs=[
            pl.BlockSpec(memory_space=pltpu.VMEM),
            pl.BlockSpec(memory_space=pltpu.VMEM),
            pl.BlockSpec(memory_space=pltpu.SMEM),
        ],
        out_specs=pl.BlockSpec(memory_space=pltpu.SMEM),
    )(c1, c2, y_true)


def kernel(y_pred, y_true):
    cands_sc = _sc_topk_candidates(y_pred)
    cands_tc = _tc_topk_candidates(y_pred)
    loss = _merge_loss(cands_sc.reshape(_NW * 5, _L),
                       cands_tc.reshape(40, 128),
                       y_true.reshape(1, 1))
    return loss.reshape(1)


# speedup vs baseline: 1.0803x; 1.0141x over previous
import jax
import jax.numpy as jnp
from jax import lax
from jax.experimental import pallas as pl
from jax.experimental.pallas import tpu as pltpu
from jax.experimental.pallas import tpu_sc as plsc

_NC = 2
_NS = 16
_L = 16
_NW = _NC * _NS

_ROWS = 128
_COLS = 32768
_SCROWS = 32
_TCROWS = _ROWS - _SCROWS
_RPW = _SCROWS // _NW
_CW = 16384
_NCHUNK = _COLS // _CW
_UNROLL = 4
_NEG = float("-inf")

_GV = 32
_CGRP = _RPW * _CW // (_GV * _L)
_CHE = _RPW * _CW


def _insert5(stack, v):
    out = []
    for t in range(5):
        hi = jnp.maximum(stack[t], v)
        v = jnp.minimum(stack[t], v)
        out.append(hi)
    return out


def _sc_body(x_hbm, out_hbm, buf0, buf1, gsum, cand, obuf, sem0, sem1):
    wid = lax.axis_index("s") * _NC + lax.axis_index("c")
    row0 = _TCROWS + wid * _RPW

    bufs = (buf0, buf1)
    sems = (sem0, sem1)

    neg = jnp.full((_L,), _NEG, dtype=jnp.float32)
    iota = lax.iota(jnp.int32, _L)

    def dyn_start(kk, h):
        for j in range(_RPW):
            pltpu.make_async_copy(
                x_hbm.at[row0 + j, pl.ds(kk * _CW, _CW)],
                bufs[h].at[pl.ds(j * _CW, _CW)], sems[h]).start()

    def dyn_wait(kk, h):
        for j in range(_RPW):
            pltpu.make_async_copy(
                x_hbm.at[row0 + j, pl.ds(kk * _CW, _CW)],
                bufs[h].at[pl.ds(j * _CW, _CW)], sems[h]).wait()

    dyn_start(0, 0)
    dyn_start(1, 1)

    def pair(it, carry):
        for h in range(2):
            kk = it * 2 + h
            buf = bufs[h]
            dyn_wait(kk, h)

            @plsc.parallel_loop(0, _CGRP, unroll=1, carry=neg)
            def sm_chunk(i, c, buf=buf):
                base = i * _GV * _L
                vs = [buf[pl.ds(base + t * _L, _L)] for t in range(_GV)]
                while len(vs) > 1:
                    vs = [jnp.maximum(vs[p], vs[p + 1])
                          for p in range(0, len(vs) - 1, 2)] + (
                              [vs[-1]] if len(vs) % 2 else [])
                gsum[pl.ds(i * _L, _L)] = vs[0]
                return jnp.maximum(c, vs[0])

            m_run = jnp.maximum(carry[0], sm_chunk)
            srt = jnp.sort(m_run)
            thr = jnp.max(jnp.where(iota == _L - 5, srt, _NEG))
            hit = jnp.any(sm_chunk >= thr)

            def docollect(_):
                def cstep(q, p):
                    m = gsum[pl.ds(q * _L, _L)]
                    h2 = jnp.any(m >= thr)
                    cand[p] = q
                    return p + h2.astype(jnp.int32)
                return lax.fori_loop(0, _CGRP, cstep, jnp.int32(0))

            p_k = lax.cond(hit, docollect, lambda _: jnp.int32(0), 0)

            def rstep(c, f, buf=buf):
                base = cand[c] * (_GV * _L)

                def ustep(u, ff, buf=buf, base=base):
                    fl = list(ff)
                    for w in range(_UNROLL):
                        v = buf[pl.ds(base + (u * _UNROLL + w) * _L, _L)]
                        fl[w * 5:(w + 1) * 5] = _insert5(
                            fl[w * 5:(w + 1) * 5], v)
                    return tuple(fl)

                return lax.fori_loop(0, _GV // _UNROLL, ustep, f)

            F = lax.fori_loop(0, p_k, rstep, carry[1:])

            @pl.when(kk + 2 < _NCHUNK)
            def _(kk=kk, h=h):
                dyn_start(kk + 2, h)

            carry = (m_run,) + tuple(F)
        return carry

    carry = lax.fori_loop(0, _NCHUNK // 2, pair,
                          (neg,) + tuple(neg for _ in range(5 * _UNROLL)))
    F = carry[1:]

    merged = list(F[0:5])
    for w in range(1, _UNROLL):
        for t in range(5):
            merged = _insert5(merged, F[w * 5 + t])

    for t in range(5):
        obuf[pl.ds(t * _L, _L)] = merged[t]
    pltpu.sync_copy(obuf, out_hbm.at[wid])


def _sc_topk_candidates(x):
    mesh = plsc.VectorSubcoreMesh(core_axis_name="c", subcore_axis_name="s",
                                  num_cores=_NC, num_subcores=_NS)
    k = pl.kernel(
        _sc_body,
        out_type=jax.ShapeDtypeStruct((_NW, 5 * _L), jnp.float32),
        mesh=mesh,
        scratch_types=[
            pltpu.VMEM((_CHE,), jnp.float32),
            pltpu.VMEM((_CHE,), jnp.float32),
            pltpu.VMEM((_CGRP * _L,), jnp.float32),
            pltpu.SMEM((_CGRP,), jnp.int32),
            pltpu.VMEM((5 * _L,), jnp.float32),
            pltpu.SemaphoreType.DMA,
            pltpu.SemaphoreType.DMA,
        ],
        compiler_params=pltpu.CompilerParams(needs_layout_passes=False),
    )
    return k(x)


_TCB = 4096
_TCG = _COLS // _TCB


def _tc_body(x_ref, o_ref, acc):
    @pl.when(pl.program_id(0) == 0)
    def _init():
        acc[...] = jnp.full((5, 8, 128), _NEG, jnp.float32)

    A = tuple(acc[t] for t in range(5))

    def cstep(c, a):
        a = list(a)
        for r in range(_TCROWS // 8):
            v = x_ref[pl.ds(r * 8, 8), pl.ds(c * 128, 128)]
            a = _insert5(a, v)
        return tuple(a)

    A = lax.fori_loop(0, _TCB // 128, cstep, A)
    for t in range(5):
        acc[t] = A[t]

    @pl.when(pl.program_id(0) == pl.num_programs(0) - 1)
    def _out():
        x = acc[...]
        li = (lax.broadcasted_iota(jnp.int32, (5, 8, 128), 0) * 1024
              + lax.broadcasted_iota(jnp.int32, (5, 8, 128), 1) * 128
              + lax.broadcasted_iota(jnp.int32, (5, 8, 128), 2))
        ri = lax.broadcasted_iota(jnp.int32, (8, 128), 0)
        ci = lax.broadcasted_iota(jnp.int32, (8, 128), 1)
        big = jnp.int32(2 ** 30)
        out = jnp.full((8, 128), _NEG, jnp.float32)
        for r in range(5):
            t = jnp.max(x)
            fi = jnp.min(jnp.where(x == t, li, big))
            x = jnp.where(li == fi, _NEG, x)
            out = jnp.where((ri == 0) & (ci == r), t, out)
        o_ref[...] = out


def _tc_topk_candidates(x):
    return pl.pallas_call(
        _tc_body,
        grid=(_TCG,),
        in_specs=[pl.BlockSpec((_TCROWS, _TCB), lambda i: (0, i))],
        out_specs=pl.BlockSpec((8, 128), lambda i: (0, 0)),
        out_shape=jax.ShapeDtypeStruct((8, 128), jnp.float32),
        scratch_shapes=[pltpu.VMEM((5, 8, 128), jnp.float32)],
    )(x)


def _merge_body(c1_ref, c2_ref, yt_ref, o_ref):
    x1 = c1_ref[...]
    x2 = c2_ref[...]
    r1, l1 = x1.shape
    r2, l2 = x2.shape
    li1 = (lax.broadcasted_iota(jnp.int32, (r1, l1), 0) * l1
           + lax.broadcasted_iota(jnp.int32, (r1, l1), 1))
    li2 = (lax.broadcasted_iota(jnp.int32, (r2, l2), 0) * l2
           + lax.broadcasted_iota(jnp.int32, (r2, l2), 1) + r1 * l1)
    big = jnp.int32(2 ** 30)
    prod = jnp.float32(1.0)
    for _ in range(5):
        t = jnp.maximum(jnp.max(x1), jnp.max(x2))
        fi = jnp.minimum(jnp.min(jnp.where(x1 == t, li1, big)),
                         jnp.min(jnp.where(x2 == t, li2, big)))
        x1 = jnp.where(li1 == fi, _NEG, x1)
        x2 = jnp.where(li2 == fi, _NEG, x2)
        prod = prod * (jnp.float32(1.0) - t)
    y_site = jnp.float32(1.0) - prod
    d = y_site - yt_ref[0, 0]
    o_ref[0, 0] = d * d


def _merge_loss(c1, c2, y_true):
    return pl.pallas_call(
        _merge_body,
        out_shape=jax.ShapeDtypeStruct((1, 1), jnp.float32),
        in_specs=[
            pl.BlockSpec(memory_space=pltpu.VMEM),
            pl.BlockSpec(memory_space=pltpu.VMEM),
            pl.BlockSpec(memory_space=pltpu.SMEM),
        ],
        out_specs=pl.BlockSpec(memory_space=pltpu.SMEM),
    )(c1, c2, y_true)


def kernel(y_pred, y_true):
    cands_sc = _sc_topk_candidates(y_pred)
    cands_tc = _tc_topk_candidates(y_pred)
    loss = _merge_loss(cands_sc, cands_tc, y_true.reshape(1, 1))
    return loss.reshape(1)
